# Initial kernel scaffold; baseline (speedup 1.0000x reference)
import jax
import jax.numpy as jnp
from jax.experimental import pallas as pl


def kernel(x, edge_index, Wq, Wk, Wv, Wo, ln1_g, ln1_b, ln2_g, ln2_b, W1, b1, W2, b2):
    raise NotImplementedError("write your pallas kernel here")


# speedup vs baseline: 11.8608x; 11.8608x over previous
import jax
import jax.numpy as jnp
from jax import lax
from jax.experimental import pallas as pl
from jax.experimental.pallas import tpu as pltpu
from jax.experimental.pallas import tpu_sc as plsc

N = 10000
E = 320000
DIM = 128
H = 4
HD = DIM // H
FF = 4 * DIM


_BR = 1000


def _qkv_body(x_ref, g_ref, b_ref, wq_ref, wk_ref, wv_ref, q_ref, k_ref, v_ref):
    xb = x_ref[...]
    mu = jnp.mean(xb, axis=1, keepdims=True)
    var = jnp.mean((xb - mu) ** 2, axis=1, keepdims=True)
    xn = (xb - mu) * lax.rsqrt(var + 1e-5) * g_ref[...] + b_ref[...]
    q_ref[...] = jnp.dot(xn, wq_ref[...], preferred_element_type=jnp.float32)
    k_ref[...] = jnp.dot(xn, wk_ref[...], preferred_element_type=jnp.float32)
    v_ref[...] = jnp.dot(xn, wv_ref[...], preferred_element_type=jnp.float32)


def _qkv_call(x, g, b, wqT, wkT, wvT):
    grid = (N // _BR,)
    return pl.pallas_call(
        _qkv_body,
        grid=grid,
        in_specs=[
            pl.BlockSpec((_BR, DIM), lambda i: (i, 0)),
            pl.BlockSpec((1, DIM), lambda i: (0, 0)),
            pl.BlockSpec((1, DIM), lambda i: (0, 0)),
            pl.BlockSpec((DIM, DIM), lambda i: (0, 0)),
            pl.BlockSpec((DIM, DIM), lambda i: (0, 0)),
            pl.BlockSpec((DIM, DIM), lambda i: (0, 0)),
        ],
        out_specs=[
            pl.BlockSpec((_BR, DIM), lambda i: (i, 0)),
            pl.BlockSpec((_BR, DIM), lambda i: (i, 0)),
            pl.BlockSpec((_BR, DIM), lambda i: (i, 0)),
        ],
        out_shape=[
            jax.ShapeDtypeStruct((N, DIM), jnp.float32),
            jax.ShapeDtypeStruct((N, DIM), jnp.float32),
            jax.ShapeDtypeStruct((N, DIM), jnp.float32),
        ],
    )(x, g.reshape(1, DIM), b.reshape(1, DIM), wqT, wkT, wvT)


_NC = 2
_NS = 16
_NW = _NC * _NS
_EPW = E // _NW
_CB = 80
_NG = _CB // 16
_NCHUNK = _EPW // _CB
_NP = 10240
_ND = _NP // 32
_RPS = _NP // _NS
_INV_SQRT_HD = 1.0 / (HD ** 0.5)


def _edge_body(q_hbm, k_hbm, v_hbm, row_hbm, col_hbm, outm_hbm, outd_hbm,
               rowi, coli, didx, qb, kb, msg, dsrc, wbuf, accm, accd,
               sem_q, sem_k):
    c = lax.axis_index("c")
    s = lax.axis_index("s")
    wid = s * _NC + c

    def _zrow(r, _):
        for j in range(DIM // 16):
            msg[r, pl.ds(16 * j, 16)] = jnp.zeros((16,), jnp.float32)
            dsrc[r, pl.ds(16 * j, 16)] = jnp.zeros((16,), jnp.float32)
        return _
    lax.fori_loop(0, _CB, _zrow, None)

    for t in range(_RPS // _CB):
        pltpu.sync_copy(msg, accm.at[pl.ds(s * _RPS + t * _CB, _CB)])

    @pl.when(s < _ND // 32)
    def _():
        pltpu.sync_copy(msg.at[pl.ds(0, 32)], accd.at[pl.ds(s * 32, 32)])

    plsc.subcore_barrier()

    def _chunk(i, _):
        eb = wid * _EPW + i * _CB
        pltpu.sync_copy(row_hbm.at[pl.ds(eb, _CB)], rowi)
        pltpu.sync_copy(col_hbm.at[pl.ds(eb, _CB)], coli)
        cq = pltpu.async_copy(q_hbm.at[rowi], qb, sem_q)
        ck = pltpu.async_copy(k_hbm.at[coli], kb, sem_k)
        cq.wait()
        ck.wait()

        def _grp(g, _):
            eids = lax.iota(jnp.int32, 16) + g * 16
            r16 = rowi[pl.ds(g * 16, 16)]
            didx[pl.ds(g * 16, 16)] = lax.shift_right_logical(r16, 5)
            cbase = lax.shift_left(jnp.bitwise_and(r16, 31), 2)
            for h in range(H):
                a = jnp.zeros((16,), jnp.float32)
                for d in range(HD):
                    dd = jnp.full((16,), 32 * h + d, jnp.int32)
                    qd = plsc.load_gather(qb, [eids, dd])
                    kd = plsc.load_gather(kb, [eids, dd])
                    a = a + qd * kd
                w = jnp.exp(a * _INV_SQRT_HD)
                plsc.store_scatter(dsrc, [eids, cbase + h], w)
                wbuf[h, pl.ds(g * 16, 16)] = w
            return _
        lax.fori_loop(0, _NG, _grp, None)

        pltpu.async_copy(v_hbm.at[coli], kb, sem_k).wait()

        def _gmsg(g, _):
            eids = lax.iota(jnp.int32, 16) + g * 16
            for h in range(H):
                w = wbuf[h, pl.ds(g * 16, 16)]
                for d in range(HD):
                    dd = jnp.full((16,), 32 * h + d, jnp.int32)
                    vd = plsc.load_gather(kb, [eids, dd])
                    plsc.store_scatter(msg, [eids, dd], vd * w)
            return _
        lax.fori_loop(0, _NG, _gmsg, None)

        pltpu.sync_copy(msg, accm.at[rowi], add=True)
        pltpu.sync_copy(dsrc, accd.at[didx], add=True)

        def _gz(g, _):
            eids = lax.iota(jnp.int32, 16) + g * 16
            r16 = rowi[pl.ds(g * 16, 16)]
            cbase = lax.shift_left(jnp.bitwise_and(r16, 31), 2)
            z = jnp.zeros((16,), jnp.float32)
            for h in range(H):
                plsc.store_scatter(dsrc, [eids, cbase + h], z)
            return _
        lax.fori_loop(0, _NG, _gz, None)
        return _
    lax.fori_loop(0, _NCHUNK, _chunk, None)

    plsc.subcore_barrier()
    for t in range(_RPS // _CB):
        r0 = s * _RPS + t * _CB
        pltpu.sync_copy(accm.at[pl.ds(r0, _CB)], outm_hbm.at[c, pl.ds(r0, _CB)])

    @pl.when(s < _ND // 32)
    def _():
        pltpu.sync_copy(accd.at[pl.ds(s * 32, 32)], outd_hbm.at[c, pl.ds(s * 32, 32)])


def _edge_call(q, k, v, row, col):
    mesh = plsc.VectorSubcoreMesh(core_axis_name="c", subcore_axis_name="s")
    f = pl.kernel(
        _edge_body,
        out_type=[
            jax.ShapeDtypeStruct((_NC, _NP, DIM), jnp.float32),
            jax.ShapeDtypeStruct((_NC, _ND, DIM), jnp.float32),
        ],
        mesh=mesh,
        scratch_types=[
            pltpu.VMEM((_CB,), jnp.int32),
            pltpu.VMEM((_CB,), jnp.int32),
            pltpu.VMEM((_CB,), jnp.int32),
            pltpu.VMEM((_CB, DIM), jnp.float32),
            pltpu.VMEM((_CB, DIM), jnp.float32),
            pltpu.VMEM((_CB, DIM), jnp.float32),
            pltpu.VMEM((_CB, DIM), jnp.float32),
            pltpu.VMEM((H, _CB), jnp.float32),
            pltpu.VMEM_SHARED((_NP, DIM), jnp.float32),
            pltpu.VMEM_SHARED((_ND, DIM), jnp.float32),
            pltpu.SemaphoreType.DMA,
            pltpu.SemaphoreType.DMA,
        ],
        compiler_params=pltpu.CompilerParams(needs_layout_passes=False),
    )
    return f(q, k, v, row, col)


def _post_body(p_ref, d_ref, x_ref, woT_ref, g2_ref, b2g_ref, w1T_ref, b1_ref,
               w2T_ref, b2_ref, o_ref):
    num = p_ref[0] + p_ref[1]
    den = d_ref[0] + d_ref[1]
    deninv = 1.0 / (den + 1e-16)
    parts = [num[:, HD * h:HD * (h + 1)] * deninv[:, h:h + 1] for h in range(H)]
    att = jnp.concatenate(parts, axis=1)
    out = jnp.dot(att, woT_ref[...], preferred_element_type=jnp.float32)
    x2 = x_ref[...] + out
    mu = jnp.mean(x2, axis=1, keepdims=True)
    var = jnp.mean((x2 - mu) ** 2, axis=1, keepdims=True)
    xn2 = (x2 - mu) * lax.rsqrt(var + 1e-5) * g2_ref[...] + b2g_ref[...]
    h1 = jnp.maximum(
        jnp.dot(xn2, w1T_ref[...], preferred_element_type=jnp.float32) + b1_ref[...],
        0.0)
    y = x2 + jnp.dot(h1, w2T_ref[...], preferred_element_type=jnp.float32) + b2_ref[...]
    o_ref[...] = y


def _post_call(p, d, x, woT, g2, b2g, w1T, b1, w2T, b2):
    grid = (N // _BR,)
    return pl.pallas_call(
        _post_body,
        grid=grid,
        in_specs=[
            pl.BlockSpec((_NC, _BR, DIM), lambda i: (0, i, 0)),
            pl.BlockSpec((_NC, _BR, H), lambda i: (0, i, 0)),
            pl.BlockSpec((_BR, DIM), lambda i: (i, 0)),
            pl.BlockSpec((DIM, DIM), lambda i: (0, 0)),
            pl.BlockSpec((1, DIM), lambda i: (0, 0)),
            pl.BlockSpec((1, DIM), lambda i: (0, 0)),
            pl.BlockSpec((DIM, FF), lambda i: (0, 0)),
            pl.BlockSpec((1, FF), lambda i: (0, 0)),
            pl.BlockSpec((FF, DIM), lambda i: (0, 0)),
            pl.BlockSpec((1, DIM), lambda i: (0, 0)),
        ],
        out_specs=pl.BlockSpec((_BR, DIM), lambda i: (i, 0)),
        out_shape=jax.ShapeDtypeStruct((N, DIM), jnp.float32),
    )(p, d, x, woT, g2.reshape(1, DIM), b2g.reshape(1, DIM), w1T,
      b1.reshape(1, FF), w2T, b2.reshape(1, DIM))


def kernel(x, edge_index, Wq, Wk, Wv, Wo, ln1_g, ln1_b, ln2_g, ln2_b, W1, b1, W2, b2):
    row = edge_index[0]
    col = edge_index[1]
    q, k, v = _qkv_call(x, ln1_g, ln1_b, Wq.T, Wk.T, Wv.T)
    pm, pd = _edge_call(q, k, v, row, col)
    den = pd.reshape(_NC, _NP, H)
    return _post_call(pm, den, x, Wo.T, ln2_g, ln2_b, W1.T, b1, W2.T, b2)
